# baseline (speedup 1.0000x reference)
import functools

import jax
import jax.numpy as jnp
from jax import lax
from jax.experimental import pallas as pl
from jax.experimental.pallas import tpu as pltpu
from jax.experimental.pallas import tpu_sc as plsc

N = 10000
NPAD = 10240
E = 320000
D = 128
DH = D // 2

NC = 2
NS = 16
EPW = E // NS
CHUNK = 80
NCHUNK = EPW // CHUNK
NBUF = 5
ROWS_PER_TILE = NPAD // NS

_mesh = plsc.VectorSubcoreMesh(core_axis_name="c", subcore_axis_name="s")


def _make_sc_agg(do_deg):
    def body(pk_hbm, x_hbm, agg_out, *rest):
        if do_deg:
            deg_out, src_v, dst_v, *rest = rest
            bufs, rest = list(rest[:NBUF]), rest[NBUF:]
            ones_buf, zbuf, zbuf16, acc_sh, deg_sh, dsem, *rest = rest
        else:
            src_v, dst_v, *rest = rest
            bufs, rest = list(rest[:NBUF]), rest[NBUF:]
            zbuf, acc_sh, *rest = rest
        gsems = list(rest)
        cid = lax.axis_index("c")
        sid = lax.axis_index("s")

        def _zrow(r, _):
            for k in range(DH // 16):
                zbuf[r, pl.ds(k * 16, 16)] = jnp.zeros((16,), jnp.float32)
            if do_deg:
                zbuf16[r, :] = jnp.zeros((16,), jnp.float32)
            return 0
        lax.fori_loop(0, 64, _zrow, 0)

        if do_deg:
            def _orow(r, _):
                ones_buf[r, :] = jnp.ones((16,), jnp.float32)
                return 0
            lax.fori_loop(0, CHUNK, _orow, 0)

        def _zacc(b, _):
            base = sid * ROWS_PER_TILE + b * 64
            pltpu.sync_copy(zbuf, acc_sh.at[pl.ds(base, 64)])
            if do_deg:
                pltpu.sync_copy(zbuf16, deg_sh.at[pl.ds(base, 64)])
            return 0
        lax.fori_loop(0, ROWS_PER_TILE // 64, _zacc, 0)

        plsc.subcore_barrier()

        pltpu.sync_copy(pk_hbm.at[sid], dst_v)

        def _xform(c):
            for k in range(CHUNK // 16):
                v = dst_v[c, pl.ds(k * 16, 16)]
                s = v & 0xFFFF
                src_v[c, pl.ds(k * 16, 16)] = s + s + cid
                dst_v[c, pl.ds(k * 16, 16)] = v >> 16

        def _xform_loop(c, _):
            _xform(c)
            return 0
        lax.fori_loop(0, NBUF, _xform_loop, 0)

        for k in range(NBUF):
            pltpu.async_copy(x_hbm.at[src_v.at[k]], bufs[k], gsems[k])

        iters = NCHUNK // NBUF

        def _ring(j, _):
            for k in range(NBUF):
                b = j * NBUF + k
                pltpu.make_async_copy(x_hbm.at[src_v.at[b]], bufs[k],
                                      gsems[k]).wait()
                if do_deg:
                    @pl.when(cid == (b % 2))
                    def _():
                        pltpu.sync_copy(ones_buf, deg_sh.at[dst_v.at[b]],
                                        add=True)
                pltpu.sync_copy(bufs[k], acc_sh.at[dst_v.at[b]], add=True)

                @pl.when(j < iters - 1)
                def _():
                    _xform(b + NBUF)
                    pltpu.async_copy(x_hbm.at[src_v.at[b + NBUF]], bufs[k],
                                     gsems[k])
            return 0
        lax.fori_loop(0, iters, _ring, 0)

        plsc.subcore_barrier()

        rows = pl.ds(sid * ROWS_PER_TILE, ROWS_PER_TILE)
        pltpu.sync_copy(acc_sh.at[rows],
                        agg_out.at[rows, pl.ds(cid * DH, DH)])
        if do_deg:
            pltpu.sync_copy(deg_sh.at[rows], deg_out.at[cid, rows])

    out_type = [jax.ShapeDtypeStruct((NPAD, D), jnp.float32)]
    if do_deg:
        out_type.append(jax.ShapeDtypeStruct((NC, NPAD, 16), jnp.float32))
    scratch = [
        pltpu.VMEM((NCHUNK, CHUNK), jnp.int32),
        pltpu.VMEM((NCHUNK, CHUNK), jnp.int32),
    ]
    scratch += [pltpu.VMEM((CHUNK, DH), jnp.float32) for _ in range(NBUF)]
    if do_deg:
        scratch.append(pltpu.VMEM((CHUNK, 16), jnp.float32))
    scratch.append(pltpu.VMEM((64, DH), jnp.float32))
    if do_deg:
        scratch.append(pltpu.VMEM((64, 16), jnp.float32))
    scratch.append(pltpu.VMEM_SHARED((NPAD, DH), jnp.float32))
    if do_deg:
        scratch.append(pltpu.VMEM_SHARED((NPAD, 16), jnp.float32))
        scratch.append(pltpu.SemaphoreType.DMA)
    scratch += [pltpu.SemaphoreType.DMA for _ in range(NBUF)]
    return pl.kernel(
        body,
        out_type=out_type,
        mesh=_mesh,
        scratch_types=scratch,
        compiler_params=pltpu.CompilerParams(use_tc_tiling_on_sc=False),
    )


_sc_agg_deg = _make_sc_agg(True)
_sc_agg = _make_sc_agg(False)


BLK = 400
GRID = N // BLK
CBLK = 2000
CGRID = N // CBLK


def _tc_self_body(x_ref, w_ref, out_ref):
    out_ref[...] = jnp.dot(x_ref[...], w_ref[...],
                           preferred_element_type=jnp.float32)


_tc_self = pl.pallas_call(
    _tc_self_body,
    grid=(GRID,),
    in_specs=[
        pl.BlockSpec((BLK, D), lambda i: (i, 0)),
        pl.BlockSpec((D, D), lambda i: (0, 0)),
    ],
    out_specs=pl.BlockSpec((BLK, D), lambda i: (i, 0)),
    out_shape=jax.ShapeDtypeStruct((N, D), jnp.float32),
)


def _tc_combine_body(relu, xs_ref, agg_ref, d0_ref, d1_ref, wn_ref,
                     out_ref, f_ref=None):
    deg = d0_ref[...] + d1_ref[...]
    inv = 1.0 / jnp.maximum(deg, 1.0)
    mean = agg_ref[...] * inv
    out = xs_ref[...] + jnp.dot(mean, wn_ref[...],
                                preferred_element_type=jnp.float32)
    if relu:
        out = jnp.maximum(out, 0.0)
    out_ref[...] = out
    if f_ref is not None:
        fm = jnp.max(out, axis=0, keepdims=True)

        @pl.when(pl.program_id(0) == 0)
        def _():
            f_ref[...] = fm

        @pl.when(pl.program_id(0) > 0)
        def _():
            f_ref[...] = jnp.maximum(f_ref[...], fm)


_combine_in_specs = [
    pl.BlockSpec((CBLK, D), lambda i: (i, 0)),
    pl.BlockSpec((CBLK, D), lambda i: (i, 0)),
    pl.BlockSpec((CBLK, 1), lambda i: (i, 0)),
    pl.BlockSpec((CBLK, 1), lambda i: (i, 0)),
    pl.BlockSpec((D, D), lambda i: (0, 0)),
]

_tc_combine1 = pl.pallas_call(
    functools.partial(_tc_combine_body, True),
    grid=(CGRID,),
    in_specs=_combine_in_specs,
    out_specs=pl.BlockSpec((CBLK, D), lambda i: (i, 0)),
    out_shape=jax.ShapeDtypeStruct((N, D), jnp.float32),
)

_tc_combine2 = pl.pallas_call(
    functools.partial(_tc_combine_body, False),
    grid=(CGRID,),
    in_specs=_combine_in_specs,
    out_specs=[
        pl.BlockSpec((CBLK, D), lambda i: (i, 0)),
        pl.BlockSpec((1, D), lambda i: (0, 0)),
    ],
    out_shape=[
        jax.ShapeDtypeStruct((N, D), jnp.float32),
        jax.ShapeDtypeStruct((1, D), jnp.float32),
    ],
)


def kernel(x, edge_index, W_self1, W_neigh1, W_self2, W_neigh2):
    ei = edge_index.astype(jnp.int32)
    packed = (jnp.left_shift(ei[1], 16) | ei[0]).reshape(NS, NCHUNK, CHUNK)

    agg, degp = _sc_agg_deg(packed, x.reshape(2 * N, DH))
    d0 = degp[0, :, 0:1]
    d1 = degp[1, :, 0:1]
    xs = _tc_self(x, W_self1)
    h = _tc_combine1(xs, agg, d0, d1, W_neigh1)

    (agg2,) = _sc_agg(packed, h.reshape(2 * N, DH))
    hs = _tc_self(h, W_self2)
    e, f = _tc_combine2(hs, agg2, d0, d1, W_neigh2)
    return (f, e)


# speedup vs baseline: 1.0306x; 1.0306x over previous
import functools

import jax
import jax.numpy as jnp
from jax import lax
from jax.experimental import pallas as pl
from jax.experimental.pallas import tpu as pltpu
from jax.experimental.pallas import tpu_sc as plsc

N = 10000
NPAD = 10240
E = 320000
D = 128
DH = D // 2

NC = 2
NS = 16
EPW = E // NS
CHUNK = 80
NCHUNK = EPW // CHUNK
NBUF = 5
ROWS_PER_TILE = NPAD // NS

_mesh = plsc.VectorSubcoreMesh(core_axis_name="c", subcore_axis_name="s")


def _make_sc_agg(do_deg):
    def body(pk_hbm, x_hbm, agg_out, *rest):
        if do_deg:
            deg_out, src_v, dst_v, *rest = rest
            bufs, rest = list(rest[:NBUF]), rest[NBUF:]
            ones_buf, zbuf, zbuf16, acc_sh, deg_sh, *rest = rest
        else:
            src_v, dst_v, *rest = rest
            bufs, rest = list(rest[:NBUF]), rest[NBUF:]
            zbuf, acc_sh, *rest = rest
        gsems = list(rest)
        cid = lax.axis_index("c")
        sid = lax.axis_index("s")

        def _zrow(r, _):
            for k in range(DH // 16):
                zbuf[r, pl.ds(k * 16, 16)] = jnp.zeros((16,), jnp.float32)
            if do_deg:
                zbuf16[r, :] = jnp.zeros((16,), jnp.float32)
            return 0
        lax.fori_loop(0, 64, _zrow, 0)

        if do_deg:
            def _orow(r, _):
                ones_buf[r, :] = jnp.ones((16,), jnp.float32)
                return 0
            lax.fori_loop(0, CHUNK, _orow, 0)

        def _zacc(b, _):
            base = sid * ROWS_PER_TILE + b * 64
            pltpu.sync_copy(zbuf, acc_sh.at[pl.ds(base, 64)])
            if do_deg:
                pltpu.sync_copy(zbuf16, deg_sh.at[pl.ds(base, 64)])
            return 0
        lax.fori_loop(0, ROWS_PER_TILE // 64, _zacc, 0)

        plsc.subcore_barrier()

        pltpu.sync_copy(pk_hbm.at[sid], dst_v)

        def _xform(c):
            for k in range(CHUNK // 16):
                v = dst_v[c, pl.ds(k * 16, 16)]
                s = v & 0xFFFF
                src_v[c, pl.ds(k * 16, 16)] = s + s + cid
                dst_v[c, pl.ds(k * 16, 16)] = v >> 16

        def _xform_loop(c, _):
            _xform(c)
            return 0
        lax.fori_loop(0, NBUF, _xform_loop, 0)

        for k in range(NBUF):
            pltpu.async_copy(x_hbm.at[src_v.at[k]], bufs[k], gsems[k])

        iters = NCHUNK // NBUF

        def _ring(j, _):
            for k in range(NBUF):
                b = j * NBUF + k
                pltpu.make_async_copy(x_hbm.at[src_v.at[b]], bufs[k],
                                      gsems[k]).wait()
                if do_deg:
                    @pl.when(cid == (b % 2))
                    def _():
                        pltpu.sync_copy(ones_buf, deg_sh.at[dst_v.at[b]],
                                        add=True)
                pltpu.sync_copy(bufs[k], acc_sh.at[dst_v.at[b]], add=True)

                @pl.when(j < iters - 1)
                def _():
                    _xform(b + NBUF)
                    pltpu.async_copy(x_hbm.at[src_v.at[b + NBUF]], bufs[k],
                                     gsems[k])
            return 0
        lax.fori_loop(0, iters, _ring, 0)

        plsc.subcore_barrier()

        rows = pl.ds(sid * ROWS_PER_TILE, ROWS_PER_TILE)
        pltpu.sync_copy(acc_sh.at[rows],
                        agg_out.at[rows, pl.ds(cid * DH, DH)])
        if do_deg:
            pltpu.sync_copy(deg_sh.at[rows], deg_out.at[cid, rows])

    out_type = [jax.ShapeDtypeStruct((NPAD, D), jnp.float32)]
    if do_deg:
        out_type.append(jax.ShapeDtypeStruct((NC, NPAD, 16), jnp.float32))
    scratch = [
        pltpu.VMEM((NCHUNK, CHUNK), jnp.int32),
        pltpu.VMEM((NCHUNK, CHUNK), jnp.int32),
    ]
    scratch += [pltpu.VMEM((CHUNK, DH), jnp.float32) for _ in range(NBUF)]
    if do_deg:
        scratch.append(pltpu.VMEM((CHUNK, 16), jnp.float32))
    scratch.append(pltpu.VMEM((64, DH), jnp.float32))
    if do_deg:
        scratch.append(pltpu.VMEM((64, 16), jnp.float32))
    scratch.append(pltpu.VMEM_SHARED((NPAD, DH), jnp.float32))
    if do_deg:
        scratch.append(pltpu.VMEM_SHARED((NPAD, 16), jnp.float32))
    scratch += [pltpu.SemaphoreType.DMA for _ in range(NBUF)]
    return pl.kernel(
        body,
        out_type=out_type,
        mesh=_mesh,
        scratch_types=scratch,
        compiler_params=pltpu.CompilerParams(use_tc_tiling_on_sc=False),
    )


_sc_agg_deg = _make_sc_agg(True)
_sc_agg = _make_sc_agg(False)


BLK = 400
GRID = N // BLK
CBLK = 2000
CGRID = N // CBLK


def _tc_self_body(x_ref, w_ref, out_ref):
    out_ref[...] = jnp.dot(x_ref[...], w_ref[...],
                           preferred_element_type=jnp.float32)


_tc_self = pl.pallas_call(
    _tc_self_body,
    grid=(GRID,),
    in_specs=[
        pl.BlockSpec((BLK, D), lambda i: (i, 0)),
        pl.BlockSpec((D, D), lambda i: (0, 0)),
    ],
    out_specs=pl.BlockSpec((BLK, D), lambda i: (i, 0)),
    out_shape=jax.ShapeDtypeStruct((N, D), jnp.float32),
)


def _tc_combine_body(relu, xs_ref, agg_ref, d0_ref, d1_ref, wn_ref,
                     out_ref, f_ref=None):
    deg16 = jnp.sum(d0_ref[0] + d1_ref[0], axis=1, keepdims=True)
    inv = 16.0 / jnp.maximum(deg16, 16.0)
    mean = agg_ref[...] * inv
    out = xs_ref[...] + jnp.dot(mean, wn_ref[...],
                                preferred_element_type=jnp.float32)
    if relu:
        out = jnp.maximum(out, 0.0)
    out_ref[...] = out
    if f_ref is not None:
        fm = jnp.max(out, axis=0, keepdims=True)

        @pl.when(pl.program_id(0) == 0)
        def _():
            f_ref[...] = fm

        @pl.when(pl.program_id(0) > 0)
        def _():
            f_ref[...] = jnp.maximum(f_ref[...], fm)


_combine_in_specs = [
    pl.BlockSpec((CBLK, D), lambda i: (i, 0)),
    pl.BlockSpec((CBLK, D), lambda i: (i, 0)),
    pl.BlockSpec((1, CBLK, 16), lambda i: (0, i, 0)),
    pl.BlockSpec((1, CBLK, 16), lambda i: (1, i, 0)),
    pl.BlockSpec((D, D), lambda i: (0, 0)),
]

_tc_combine1 = pl.pallas_call(
    functools.partial(_tc_combine_body, True),
    grid=(CGRID,),
    in_specs=_combine_in_specs,
    out_specs=pl.BlockSpec((CBLK, D), lambda i: (i, 0)),
    out_shape=jax.ShapeDtypeStruct((N, D), jnp.float32),
)

_tc_combine2 = pl.pallas_call(
    functools.partial(_tc_combine_body, False),
    grid=(CGRID,),
    in_specs=_combine_in_specs,
    out_specs=[
        pl.BlockSpec((CBLK, D), lambda i: (i, 0)),
        pl.BlockSpec((1, D), lambda i: (0, 0)),
    ],
    out_shape=[
        jax.ShapeDtypeStruct((N, D), jnp.float32),
        jax.ShapeDtypeStruct((1, D), jnp.float32),
    ],
)


def kernel(x, edge_index, W_self1, W_neigh1, W_self2, W_neigh2):
    ei = edge_index.astype(jnp.int32)
    packed = (jnp.left_shift(ei[1], 16) | ei[0]).reshape(NS, NCHUNK, CHUNK)

    agg, degp = _sc_agg_deg(packed, x.reshape(2 * N, DH))
    xs = _tc_self(x, W_self1)
    h = _tc_combine1(xs, agg, degp, degp, W_neigh1)

    (agg2,) = _sc_agg(packed, h.reshape(2 * N, DH))
    hs = _tc_self(h, W_self2)
    e, f = _tc_combine2(hs, agg2, degp, degp, W_neigh2)
    return (f, e)
